# baseline (speedup 1.0000x reference)
import functools

import jax
import jax.numpy as jnp
from jax import lax
from jax.experimental import pallas as pl
from jax.experimental.pallas import tpu as pltpu
from jax.experimental.pallas import tpu_sc as plsc

_INFO = plsc.get_sparse_core_info()
_NC, _NS, _L = _INFO.num_cores, _INFO.num_subcores, _INFO.num_lanes
_NW = _NC * _NS


@functools.partial(jax.jit, static_argnames=("rows", "cols", "nidx"))
def _gather_elements(in_flat, idx_flat, *, rows, cols, nidx):
    rows_per_w = rows // _NW
    chunk = 32
    half = chunk // 2
    nbuf = 2
    n_chunks = rows_per_w // chunk
    pair_vecs = (2 * nidx) // _L

    mesh = plsc.VectorSubcoreMesh(core_axis_name="c", subcore_axis_name="s")

    @functools.partial(
        pl.kernel,
        out_type=jax.ShapeDtypeStruct((rows * nidx,), jnp.float32),
        mesh=mesh,
        compiler_params=pltpu.CompilerParams(needs_layout_passes=False),
        scratch_types=[
            [pltpu.VMEM((chunk * cols,), jnp.float32) for _ in range(nbuf)],
            [pltpu.VMEM((chunk * nidx,), jnp.int32) for _ in range(nbuf)],
            [pltpu.VMEM((chunk * nidx,), jnp.float32) for _ in range(nbuf)],
            [pltpu.SemaphoreType.DMA for _ in range(5 * nbuf)],
        ],
    )
    def k(in_hbm, idx_hbm, out_hbm, in_v, idx_v, out_v, sems):
        wid = lax.axis_index("s") * _NC + lax.axis_index("c")
        w_row0 = wid * rows_per_w
        iota = lax.broadcasted_iota(jnp.int32, (_L,), 0)

        def start_in(c, b, h):
            row0 = w_row0 + c * chunk + h * half
            pltpu.async_copy(
                in_hbm.at[pl.ds(row0 * cols, half * cols)],
                in_v[b].at[pl.ds(h * half * cols, half * cols)],
                sems[2 * h + b])
            pltpu.async_copy(
                idx_hbm.at[pl.ds(row0 * nidx, half * nidx)],
                idx_v[b].at[pl.ds(h * half * nidx, half * nidx)],
                sems[4 + 2 * h + b])

        def wait_in(b, h):
            pltpu.make_async_copy(
                in_hbm.at[pl.ds(0, half * cols)],
                in_v[b].at[pl.ds(h * half * cols, half * cols)],
                sems[2 * h + b]).wait()
            pltpu.make_async_copy(
                idx_hbm.at[pl.ds(0, half * nidx)],
                idx_v[b].at[pl.ds(h * half * nidx, half * nidx)],
                sems[4 + 2 * h + b]).wait()

        def start_out(c, b):
            row0 = w_row0 + c * chunk
            pltpu.async_copy(
                out_v[b], out_hbm.at[pl.ds(row0 * nidx, chunk * nidx)],
                sems[8 + b])

        def wait_out(b):
            pltpu.make_async_copy(
                out_v[b], out_hbm.at[pl.ds(0, chunk * nidx)],
                sems[8 + b]).wait()

        def compute(b, h):
            iv, xv, ov = in_v[b], idx_v[b], out_v[b]

            @plsc.parallel_loop(h * half // 2, (h + 1) * half // 2, unroll=1)
            def pair_body(p):
                fbase = p * (2 * cols)
                for v in range(pair_vecs):
                    base = p * (2 * nidx) + v * _L
                    roff = jnp.where((v * _L + iota) >= nidx, cols, 0)
                    colv = xv[pl.ds(base, _L)]
                    ov[pl.ds(base, _L)] = plsc.load_gather(
                        iv, [colv + fbase + roff])

        def do_chunk(c, b, first):
            wait_in(b, 0)
            if not first:
                wait_out(b)
            compute(b, 0)
            wait_in(b, 1)
            compute(b, 1)
            if not isinstance(c, int) or c + nbuf < n_chunks:
                start_in(c + nbuf, b, 0)
                start_in(c + nbuf, b, 1)
            start_out(c, b)

        for b in range(nbuf):
            start_in(b, b, 0)
            start_in(b, b, 1)
        for b in range(nbuf):
            do_chunk(b, b, first=True)

        def super_body(g, _):
            c0 = nbuf * g
            for b in range(nbuf):
                do_chunk(c0 + b, b, first=False)
            return 0

        lax.fori_loop(1, n_chunks // nbuf - 1, super_body, 0)

        for b in range(nbuf):
            c = n_chunks - nbuf + b
            wait_in(b, 0)
            wait_out(b)
            compute(b, 0)
            wait_in(b, 1)
            compute(b, 1)
            start_out(c, b)
        for b in range(nbuf):
            wait_out(b)

    return k(in_flat, idx_flat)


def kernel(input_tensor, indices):
    rows, cols = input_tensor.shape
    nidx = indices.shape[1]
    in_flat = input_tensor.reshape(-1)
    idx_flat = indices.astype(jnp.int32).reshape(-1)
    out = _gather_elements(in_flat, idx_flat, rows=rows, cols=cols, nidx=nidx)
    return out.reshape(rows, nidx)


# speedup vs baseline: 1.0255x; 1.0255x over previous
import functools

import jax
import jax.numpy as jnp
from jax import lax
from jax.experimental import pallas as pl
from jax.experimental.pallas import tpu as pltpu
from jax.experimental.pallas import tpu_sc as plsc

_INFO = plsc.get_sparse_core_info()
_NC, _NS, _L = _INFO.num_cores, _INFO.num_subcores, _INFO.num_lanes
_NW = _NC * _NS


@functools.partial(jax.jit, static_argnames=("rows", "cols", "nidx"))
def _gather_elements(in_flat, idx_flat, *, rows, cols, nidx):
    rows_per_w = rows // _NW
    chunk = 32
    nbuf = 2
    n_chunks = rows_per_w // chunk
    pair_vecs = (2 * nidx) // _L

    mesh = plsc.VectorSubcoreMesh(core_axis_name="c", subcore_axis_name="s")

    @functools.partial(
        pl.kernel,
        out_type=jax.ShapeDtypeStruct((rows * nidx,), jnp.float32),
        mesh=mesh,
        compiler_params=pltpu.CompilerParams(needs_layout_passes=False),
        scratch_types=[
            [pltpu.VMEM((chunk * cols,), jnp.float32) for _ in range(nbuf)],
            [pltpu.VMEM((chunk * nidx,), jnp.int32) for _ in range(nbuf)],
            [pltpu.VMEM((chunk * nidx,), jnp.float32) for _ in range(nbuf)],
            [pltpu.SemaphoreType.DMA for _ in range(3 * nbuf)],
        ],
    )
    def k(in_hbm, idx_hbm, out_hbm, in_v, idx_v, out_v, sems):
        wid = lax.axis_index("s") * _NC + lax.axis_index("c")
        w_row0 = wid * rows_per_w
        iota = lax.broadcasted_iota(jnp.int32, (_L,), 0)

        def start_in(c, b):
            row0 = w_row0 + c * chunk
            pltpu.async_copy(
                in_hbm.at[pl.ds(row0 * cols, chunk * cols)], in_v[b], sems[b])
            pltpu.async_copy(
                idx_hbm.at[pl.ds(row0 * nidx, chunk * nidx)], idx_v[b],
                sems[nbuf + b])

        def wait_in(b):
            pltpu.make_async_copy(
                in_hbm.at[pl.ds(0, chunk * cols)], in_v[b], sems[b]).wait()
            pltpu.make_async_copy(
                idx_hbm.at[pl.ds(0, chunk * nidx)], idx_v[b],
                sems[nbuf + b]).wait()

        def start_out(c, b):
            row0 = w_row0 + c * chunk
            pltpu.async_copy(
                out_v[b], out_hbm.at[pl.ds(row0 * nidx, chunk * nidx)],
                sems[2 * nbuf + b])

        def wait_out(b):
            pltpu.make_async_copy(
                out_v[b], out_hbm.at[pl.ds(0, chunk * nidx)],
                sems[2 * nbuf + b]).wait()

        def compute(b):
            iv, xv, ov = in_v[b], idx_v[b], out_v[b]

            @plsc.parallel_loop(0, chunk // 2, unroll=1)
            def pair_body(p):
                fbase = p * (2 * cols)
                for v in range(pair_vecs):
                    base = p * (2 * nidx) + v * _L
                    roff = jnp.where((v * _L + iota) >= nidx, cols, 0)
                    colv = xv[pl.ds(base, _L)]
                    ov[pl.ds(base, _L)] = plsc.load_gather(
                        iv, [colv + fbase + roff])

        for b in range(nbuf):
            start_in(b, b)
        for b in range(nbuf):
            wait_in(b)
            compute(b)
            start_in(b + nbuf, b)
            start_out(b, b)

        def super_body(g, _):
            c0 = nbuf * g
            for b in range(nbuf):
                wait_in(b)
                wait_out(b)
                compute(b)
                start_in(c0 + b + nbuf, b)
                start_out(c0 + b, b)
            return 0

        lax.fori_loop(1, n_chunks // nbuf - 1, super_body, 0)

        for b in range(nbuf):
            wait_in(b)
            wait_out(b)
            compute(b)
            start_out(n_chunks - nbuf + b, b)
        for b in range(nbuf):
            wait_out(b)

    return k(in_flat, idx_flat)


def kernel(input_tensor, indices):
    rows, cols = input_tensor.shape
    nidx = indices.shape[1]
    in_flat = input_tensor.reshape(-1)
    idx_flat = indices.astype(jnp.int32).reshape(-1)
    out = _gather_elements(in_flat, idx_flat, rows=rows, cols=cols, nidx=nidx)
    return out.reshape(rows, nidx)
